# baseline (device time: 20466 ns/iter reference)
import jax
import jax.numpy as jnp
from jax import lax
from jax.experimental import pallas as pl
from jax.experimental.pallas import tpu as pltpu

N_ROW_BLOCKS = 8


def kernel(x, dy, gamma):
    m, d = x.shape
    blk = m // N_ROW_BLOCKS

    def body(x_ref, dy_ref, out_ref, acc_ref, comm_ref, send_sem, recv_sem):
        i = pl.program_id(0)

        @pl.when(i == 0)
        def _():
            acc_ref[...] = jnp.zeros_like(acc_ref)

        xv = x_ref[...]
        dyv = dy_ref[...]
        mu = jnp.mean(xv, axis=1, keepdims=True)
        xc = xv - mu
        var = jnp.mean(xc * xc, axis=1, keepdims=True)
        xhat = xc * lax.rsqrt(var + 1e-5)
        dg = jnp.sum(dyv * xhat, axis=0, keepdims=True)
        db = jnp.sum(dyv, axis=0, keepdims=True)
        acc_ref[...] += jnp.concatenate([dg, db], axis=0)

        @pl.when(i == N_ROW_BLOCKS - 1)
        def _():
            my_x = lax.axis_index("x")
            my_y = lax.axis_index("y")
            nbr = (my_x, 1 - my_y)

            barrier_sem = pltpu.get_barrier_semaphore()
            pl.semaphore_signal(
                barrier_sem, inc=1,
                device_id=nbr, device_id_type=pl.DeviceIdType.MESH,
            )
            pl.semaphore_wait(barrier_sem, 1)

            rdma = pltpu.make_async_remote_copy(
                src_ref=acc_ref,
                dst_ref=comm_ref,
                send_sem=send_sem,
                recv_sem=recv_sem,
                device_id=nbr,
                device_id_type=pl.DeviceIdType.MESH,
            )
            rdma.start()
            rdma.wait()

            out_ref[...] = acc_ref[...] + comm_ref[...]

    return pl.pallas_call(
        body,
        grid=(N_ROW_BLOCKS,),
        in_specs=[
            pl.BlockSpec((blk, d), lambda i: (i, 0)),
            pl.BlockSpec((blk, d), lambda i: (i, 0)),
        ],
        out_specs=pl.BlockSpec((2, d), lambda i: (0, 0)),
        out_shape=jax.ShapeDtypeStruct((2, d), jnp.float32),
        scratch_shapes=[
            pltpu.VMEM((2, d), jnp.float32),
            pltpu.VMEM((2, d), jnp.float32),
            pltpu.SemaphoreType.DMA,
            pltpu.SemaphoreType.DMA,
        ],
        compiler_params=pltpu.CompilerParams(collective_id=0),
    )(x, dy)


# device time: 16818 ns/iter; 1.2169x vs baseline; 1.2169x over previous
import jax
import jax.numpy as jnp
from jax import lax
from jax.experimental import pallas as pl
from jax.experimental.pallas import tpu as pltpu

N_ROW_BLOCKS = 4


def kernel(x, dy, gamma):
    m, d = x.shape
    half = m // 2
    blk = half // N_ROW_BLOCKS

    def body(mx_ref, x_ref, dy_ref, out_ref, acc_ref, comm_ref,
             send_sems, recv_sems):
        i = pl.program_id(0)

        @pl.when(i == 0)
        def _():
            acc_ref[...] = jnp.zeros_like(acc_ref)
            my_x0 = mx_ref[0]
            my_y0 = lax.axis_index("y")
            barrier_sem0 = pltpu.get_barrier_semaphore()
            for p in [(my_x0, 1 - my_y0), (1 - my_x0, my_y0),
                      (1 - my_x0, 1 - my_y0)]:
                pl.semaphore_signal(
                    barrier_sem0, inc=1,
                    device_id=p, device_id_type=pl.DeviceIdType.MESH,
                )

        xv = x_ref[...]
        dyv = dy_ref[...]
        mu = jnp.mean(xv, axis=1, keepdims=True)
        xc = xv - mu
        var = jnp.mean(xc * xc, axis=1, keepdims=True)
        xhat = xc * lax.rsqrt(var + 1e-5)
        dg = jnp.sum(dyv * xhat, axis=0, keepdims=True)
        db = jnp.sum(dyv, axis=0, keepdims=True)
        acc_ref[...] += jnp.concatenate([dg, db], axis=0)

        @pl.when(i == N_ROW_BLOCKS - 1)
        def _():
            my_x = mx_ref[0]
            my_y = lax.axis_index("y")
            peers = [
                (my_x, 1 - my_y),
                (1 - my_x, my_y),
                (1 - my_x, 1 - my_y),
            ]

            barrier_sem = pltpu.get_barrier_semaphore()
            pl.semaphore_wait(barrier_sem, len(peers))

            rdmas = []
            for slot, p in enumerate(peers):
                rdma = pltpu.make_async_remote_copy(
                    src_ref=acc_ref,
                    dst_ref=comm_ref.at[slot],
                    send_sem=send_sems.at[slot],
                    recv_sem=recv_sems.at[slot],
                    device_id=p,
                    device_id_type=pl.DeviceIdType.MESH,
                )
                rdma.start()
                rdmas.append(rdma)
            for rdma in rdmas:
                rdma.wait()

            out_ref[...] = (
                acc_ref[...] + comm_ref[0] + comm_ref[1] + comm_ref[2]
            )

    my_x = lax.axis_index("x").astype(jnp.int32)

    grid_spec = pltpu.PrefetchScalarGridSpec(
        num_scalar_prefetch=1,
        grid=(N_ROW_BLOCKS,),
        in_specs=[
            pl.BlockSpec((blk, d), lambda i, mx: (mx[0] * N_ROW_BLOCKS + i, 0)),
            pl.BlockSpec((blk, d), lambda i, mx: (mx[0] * N_ROW_BLOCKS + i, 0)),
        ],
        out_specs=pl.BlockSpec((2, d), lambda i, mx: (0, 0)),
        scratch_shapes=[
            pltpu.VMEM((2, d), jnp.float32),
            pltpu.VMEM((3, 2, d), jnp.float32),
            pltpu.SemaphoreType.DMA((3,)),
            pltpu.SemaphoreType.DMA((3,)),
        ],
    )

    return pl.pallas_call(
        body,
        grid_spec=grid_spec,
        out_shape=jax.ShapeDtypeStruct((2, d), jnp.float32),
        compiler_params=pltpu.CompilerParams(collective_id=0),
    )(jnp.reshape(my_x, (1,)), x, dy)


# device time: 16466 ns/iter; 1.2429x vs baseline; 1.0214x over previous
import jax
import jax.numpy as jnp
from jax import lax
from jax.experimental import pallas as pl
from jax.experimental.pallas import tpu as pltpu

N_ROW_BLOCKS = 2


def kernel(x, dy, gamma):
    m, d = x.shape
    half = m // 2
    blk = half // N_ROW_BLOCKS

    def body(mx_ref, x_ref, dy_ref, out_ref, acc_ref, comm_ref,
             send_sems, recv_sems):
        i = pl.program_id(0)

        @pl.when(i == 0)
        def _():
            acc_ref[...] = jnp.zeros_like(acc_ref)
            my_x0 = mx_ref[0]
            my_y0 = lax.axis_index("y")
            barrier_sem0 = pltpu.get_barrier_semaphore()
            for p in [(my_x0, 1 - my_y0), (1 - my_x0, my_y0),
                      (1 - my_x0, 1 - my_y0)]:
                pl.semaphore_signal(
                    barrier_sem0, inc=1,
                    device_id=p, device_id_type=pl.DeviceIdType.MESH,
                )

        xv = x_ref[...]
        dyv = dy_ref[...]
        mu = jnp.mean(xv, axis=1, keepdims=True)
        xc = xv - mu
        var = jnp.mean(xc * xc, axis=1, keepdims=True)
        xhat = xc * lax.rsqrt(var + 1e-5)
        dg = jnp.sum(dyv * xhat, axis=0, keepdims=True)
        db = jnp.sum(dyv, axis=0, keepdims=True)
        acc_ref[...] += jnp.concatenate([dg, db], axis=0)

        @pl.when(i == N_ROW_BLOCKS - 1)
        def _():
            my_x = mx_ref[0]
            my_y = lax.axis_index("y")
            peers = [
                (my_x, 1 - my_y),
                (1 - my_x, my_y),
                (1 - my_x, 1 - my_y),
            ]

            barrier_sem = pltpu.get_barrier_semaphore()
            pl.semaphore_wait(barrier_sem, len(peers))

            rdmas = []
            for slot, p in enumerate(peers):
                rdma = pltpu.make_async_remote_copy(
                    src_ref=acc_ref,
                    dst_ref=comm_ref.at[slot],
                    send_sem=send_sems.at[slot],
                    recv_sem=recv_sems.at[slot],
                    device_id=p,
                    device_id_type=pl.DeviceIdType.MESH,
                )
                rdma.start()
                rdmas.append(rdma)
            for rdma in rdmas:
                rdma.wait()

            out_ref[...] = (
                acc_ref[...] + comm_ref[0] + comm_ref[1] + comm_ref[2]
            )

    my_x = lax.axis_index("x").astype(jnp.int32)

    grid_spec = pltpu.PrefetchScalarGridSpec(
        num_scalar_prefetch=1,
        grid=(N_ROW_BLOCKS,),
        in_specs=[
            pl.BlockSpec((blk, d), lambda i, mx: (mx[0] * N_ROW_BLOCKS + i, 0)),
            pl.BlockSpec((blk, d), lambda i, mx: (mx[0] * N_ROW_BLOCKS + i, 0)),
        ],
        out_specs=pl.BlockSpec((2, d), lambda i, mx: (0, 0)),
        scratch_shapes=[
            pltpu.VMEM((2, d), jnp.float32),
            pltpu.VMEM((3, 2, d), jnp.float32),
            pltpu.SemaphoreType.DMA((3,)),
            pltpu.SemaphoreType.DMA((3,)),
        ],
    )

    return pl.pallas_call(
        body,
        grid_spec=grid_spec,
        out_shape=jax.ShapeDtypeStruct((2, d), jnp.float32),
        compiler_params=pltpu.CompilerParams(collective_id=0),
    )(jnp.reshape(my_x, (1,)), x, dy)
